# initial kernel scaffold (unmeasured)
import jax
import jax.numpy as jnp
from jax import lax
from jax.experimental import pallas as pl
from jax.experimental.pallas import tpu as pltpu

N_DEV = 8


def kernel(x, w_mat):
    m, k_loc = x.shape
    _, n = w_mat.shape
    m_chunk = m // N_DEV

    def body(x_ref, w_ref, out_ref, comm_ref, amax_ref,
             send_sems, recv_sems, amax_send_sems, amax_recv_sems,
             credit_sem):
        d = lax.axis_index("i")
        left = lax.rem(d + N_DEV - 1, N_DEV)
        right = lax.rem(d + 1, N_DEV)

        barrier_sem = pltpu.get_barrier_semaphore()
        for nbr in (left, right):
            pl.semaphore_signal(barrier_sem, inc=1, device_id=(nbr,),
                                device_id_type=pl.DeviceIdType.MESH)
        pl.semaphore_wait(barrier_sem, 2)

        def partial_chunk(c):
            return jnp.dot(x_ref[pl.ds(c * m_chunk, m_chunk), :], w_ref[:, :],
                           preferred_element_type=jnp.float32)

        comm_ref[0] = partial_chunk(lax.rem(d + N_DEV - 1, N_DEV))

        for s in range(N_DEV - 1):
            send_slot = s % 2
            recv_slot = (s + 1) % 2
            if s >= 1:
                pl.semaphore_wait(credit_sem, 1)
            rdma = pltpu.make_async_remote_copy(
                src_ref=comm_ref.at[send_slot],
                dst_ref=comm_ref.at[recv_slot],
                send_sem=send_sems.at[send_slot],
                recv_sem=recv_sems.at[recv_slot],
                device_id=(right,),
                device_id_type=pl.DeviceIdType.MESH,
            )
            rdma.start()
            rdma.wait()
            if s < N_DEV - 2:
                pl.semaphore_signal(credit_sem, inc=1, device_id=(left,),
                                    device_id_type=pl.DeviceIdType.MESH)
            c = lax.rem(d + 2 * N_DEV - s - 2, N_DEV)
            if s < N_DEV - 2:
                comm_ref[recv_slot] += partial_chunk(c)
            else:
                out_ref[...] = jnp.maximum(
                    comm_ref[recv_slot] + partial_chunk(c), 0.0)

        local_amax = jnp.max(out_ref[...])
        amax_ref[pl.ds(d, 1)] = jnp.full((1, 8, 128), local_amax,
                                         dtype=jnp.float32)
        send_rdmas = []
        for off in range(1, N_DEV):
            peer = lax.rem(d + off, N_DEV)
            r = pltpu.make_async_remote_copy(
                src_ref=amax_ref.at[d],
                dst_ref=amax_ref.at[d],
                send_sem=amax_send_sems.at[off - 1],
                recv_sem=amax_recv_sems.at[off - 1],
                device_id=(peer,),
                device_id_type=pl.DeviceIdType.MESH,
            )
            r.start()
            send_rdmas.append(r)
        for j in range(1, N_DEV):
            src_dev = lax.rem(d + N_DEV - j, N_DEV)
            recv = pltpu.make_async_remote_copy(
                src_ref=amax_ref.at[src_dev],
                dst_ref=amax_ref.at[src_dev],
                send_sem=amax_send_sems.at[j - 1],
                recv_sem=amax_recv_sems.at[j - 1],
                device_id=(src_dev,),
                device_id_type=pl.DeviceIdType.MESH,
            )
            recv.wait_recv()
        for r in send_rdmas:
            r.wait_send()

        amax = jnp.max(amax_ref[...])
        scale = jnp.where(amax > 0.0, amax / 448.0, 1.0)
        q = (out_ref[...] / scale).astype(jnp.float8_e4m3fn)
        out_ref[...] = q.astype(jnp.float32) * scale

    return pl.pallas_call(
        body,
        out_shape=jax.ShapeDtypeStruct((m_chunk, n), jnp.float32),
        in_specs=[
            pl.BlockSpec(memory_space=pltpu.VMEM),
            pl.BlockSpec(memory_space=pltpu.VMEM),
        ],
        out_specs=pl.BlockSpec(memory_space=pltpu.VMEM),
        scratch_shapes=[
            pltpu.VMEM((2, m_chunk, n), jnp.float32),
            pltpu.VMEM((N_DEV, 8, 128), jnp.float32),
            pltpu.SemaphoreType.DMA((2,)),
            pltpu.SemaphoreType.DMA((2,)),
            pltpu.SemaphoreType.DMA((N_DEV - 1,)),
            pltpu.SemaphoreType.DMA((N_DEV - 1,)),
            pltpu.SemaphoreType.REGULAR,
        ],
        compiler_params=pltpu.CompilerParams(collective_id=0),
    )(x, w_mat)


# baseline (device time: 1396458 ns/iter reference)
import jax
import jax.numpy as jnp
from jax import lax
from jax.experimental import pallas as pl
from jax.experimental.pallas import tpu as pltpu

N_DEV = 8
G = 4


def kernel(x, w_mat):
    m, k_loc = x.shape
    _, n = w_mat.shape
    m_chunk = m // N_DEV
    w_g = n // G

    def body(x_ref, w_ref, out_ref, comm_ref, amax_ref,
             send_sems, recv_sems, amax_send_sems, amax_recv_sems,
             credit_sem):
        d = lax.axis_index("i")
        left = lax.rem(d + N_DEV - 1, N_DEV)
        right = lax.rem(d + 1, N_DEV)

        barrier_sem = pltpu.get_barrier_semaphore()
        for nbr in (left, right):
            pl.semaphore_signal(barrier_sem, inc=1, device_id=(nbr,),
                                device_id_type=pl.DeviceIdType.MESH)
        pl.semaphore_wait(barrier_sem, 2)

        def partial(c, g):
            return jnp.dot(x_ref[pl.ds(c * m_chunk, m_chunk), :],
                           w_ref[:, g * w_g:(g + 1) * w_g],
                           preferred_element_type=jnp.float32)

        n_steps = G * (N_DEV - 1)
        t = 0
        for g in range(G):
            comm_ref[t % 2] = partial(lax.rem(d + N_DEV - 1, N_DEV), g)
            for s in range(N_DEV - 1):
                send_slot = t % 2
                recv_slot = (t + 1) % 2
                if t >= 1:
                    pl.semaphore_wait(credit_sem, 1)
                rdma = pltpu.make_async_remote_copy(
                    src_ref=comm_ref.at[send_slot],
                    dst_ref=comm_ref.at[recv_slot],
                    send_sem=send_sems.at[send_slot],
                    recv_sem=recv_sems.at[recv_slot],
                    device_id=(right,),
                    device_id_type=pl.DeviceIdType.MESH,
                )
                rdma.start()
                rdma.wait()
                if t < n_steps - 1:
                    pl.semaphore_signal(credit_sem, inc=1, device_id=(left,),
                                        device_id_type=pl.DeviceIdType.MESH)
                c = lax.rem(d + 2 * N_DEV - s - 2, N_DEV)
                if s < N_DEV - 2:
                    comm_ref[recv_slot] += partial(c, g)
                else:
                    out_ref[:, g * w_g:(g + 1) * w_g] = jnp.maximum(
                        comm_ref[recv_slot] + partial(c, g), 0.0)
                t += 1

        local_amax = jnp.max(out_ref[...])
        amax_ref[pl.ds(d, 1)] = jnp.full((1, 8, 128), local_amax,
                                         dtype=jnp.float32)
        send_rdmas = []
        for off in range(1, N_DEV):
            peer = lax.rem(d + off, N_DEV)
            r = pltpu.make_async_remote_copy(
                src_ref=amax_ref.at[d],
                dst_ref=amax_ref.at[d],
                send_sem=amax_send_sems.at[off - 1],
                recv_sem=amax_recv_sems.at[off - 1],
                device_id=(peer,),
                device_id_type=pl.DeviceIdType.MESH,
            )
            r.start()
            send_rdmas.append(r)
        for j in range(1, N_DEV):
            src_dev = lax.rem(d + N_DEV - j, N_DEV)
            recv = pltpu.make_async_remote_copy(
                src_ref=amax_ref.at[src_dev],
                dst_ref=amax_ref.at[src_dev],
                send_sem=amax_send_sems.at[j - 1],
                recv_sem=amax_recv_sems.at[j - 1],
                device_id=(src_dev,),
                device_id_type=pl.DeviceIdType.MESH,
            )
            recv.wait_recv()
        for r in send_rdmas:
            r.wait_send()

        amax = jnp.max(amax_ref[...])
        scale = jnp.where(amax > 0.0, amax / 448.0, 1.0)
        for g in range(G):
            cols = pl.ds(g * w_g, w_g)
            q = (out_ref[:, cols] / scale).astype(jnp.float8_e4m3fn)
            out_ref[:, cols] = q.astype(jnp.float32) * scale

    return pl.pallas_call(
        body,
        out_shape=jax.ShapeDtypeStruct((m_chunk, n), jnp.float32),
        in_specs=[
            pl.BlockSpec(memory_space=pltpu.VMEM),
            pl.BlockSpec(memory_space=pltpu.VMEM),
        ],
        out_specs=pl.BlockSpec(memory_space=pltpu.VMEM),
        scratch_shapes=[
            pltpu.VMEM((2, m_chunk, w_g), jnp.float32),
            pltpu.VMEM((N_DEV, 8, 128), jnp.float32),
            pltpu.SemaphoreType.DMA((2,)),
            pltpu.SemaphoreType.DMA((2,)),
            pltpu.SemaphoreType.DMA((N_DEV - 1,)),
            pltpu.SemaphoreType.DMA((N_DEV - 1,)),
            pltpu.SemaphoreType.REGULAR,
        ],
        compiler_params=pltpu.CompilerParams(
            collective_id=0, vmem_limit_bytes=60 * 1024 * 1024),
    )(x, w_mat)


# device time: 740816 ns/iter; 1.8850x vs baseline; 1.8850x over previous
import jax
import jax.numpy as jnp
from jax import lax
from jax.experimental import pallas as pl
from jax.experimental.pallas import tpu as pltpu

N_DEV = 8
G = 4
N_EPOCH = G // 2


def kernel(x, w_mat):
    m, k_loc = x.shape
    _, n = w_mat.shape
    m_chunk = m // N_DEV
    w_g = n // G

    def body(x_ref, w_ref, out_ref, comm_a, comm_b, amax_ref,
             send_sems_a, recv_sems_a, send_sems_b, recv_sems_b,
             amax_send_sems, amax_recv_sems, credit_a, credit_b):
        d = lax.axis_index("i")
        left = lax.rem(d + N_DEV - 1, N_DEV)
        right = lax.rem(d + 1, N_DEV)

        barrier_sem = pltpu.get_barrier_semaphore()
        for nbr in (left, right):
            pl.semaphore_signal(barrier_sem, inc=1, device_id=(nbr,),
                                device_id_type=pl.DeviceIdType.MESH)
        pl.semaphore_wait(barrier_sem, 2)

        def partial(c, g):
            return jnp.dot(x_ref[pl.ds(c * m_chunk, m_chunk), :],
                           w_ref[:, g * w_g:(g + 1) * w_g],
                           preferred_element_type=jnp.float32)

        n_steps = N_EPOCH * (N_DEV - 1)
        t = 0
        for e in range(N_EPOCH):
            ga, gb = 2 * e, 2 * e + 1
            comm_a[t % 2] = partial(lax.rem(d + N_DEV - 1, N_DEV), ga)
            comm_b[t % 2] = partial(lax.rem(d + 1, N_DEV), gb)
            for s in range(N_DEV - 1):
                send_slot = t % 2
                recv_slot = (t + 1) % 2
                if t >= 1:
                    pl.semaphore_wait(credit_a, 1)
                    pl.semaphore_wait(credit_b, 1)
                rdma_a = pltpu.make_async_remote_copy(
                    src_ref=comm_a.at[send_slot],
                    dst_ref=comm_a.at[recv_slot],
                    send_sem=send_sems_a.at[send_slot],
                    recv_sem=recv_sems_a.at[recv_slot],
                    device_id=(right,),
                    device_id_type=pl.DeviceIdType.MESH,
                )
                rdma_b = pltpu.make_async_remote_copy(
                    src_ref=comm_b.at[send_slot],
                    dst_ref=comm_b.at[recv_slot],
                    send_sem=send_sems_b.at[send_slot],
                    recv_sem=recv_sems_b.at[recv_slot],
                    device_id=(left,),
                    device_id_type=pl.DeviceIdType.MESH,
                )
                rdma_a.start()
                rdma_b.start()
                rdma_a.wait()
                rdma_b.wait()
                if t < n_steps - 1:
                    pl.semaphore_signal(credit_a, inc=1, device_id=(left,),
                                        device_id_type=pl.DeviceIdType.MESH)
                    pl.semaphore_signal(credit_b, inc=1, device_id=(right,),
                                        device_id_type=pl.DeviceIdType.MESH)
                ca = lax.rem(d + 2 * N_DEV - s - 2, N_DEV)
                cb = lax.rem(d + s + 2, N_DEV)
                if s < N_DEV - 2:
                    comm_a[recv_slot] += partial(ca, ga)
                    comm_b[recv_slot] += partial(cb, gb)
                else:
                    out_ref[:, ga * w_g:(ga + 1) * w_g] = jnp.maximum(
                        comm_a[recv_slot] + partial(ca, ga), 0.0)
                    out_ref[:, gb * w_g:(gb + 1) * w_g] = jnp.maximum(
                        comm_b[recv_slot] + partial(cb, gb), 0.0)
                t += 1

        local_amax = jnp.max(out_ref[...])
        amax_ref[pl.ds(d, 1)] = jnp.full((1, 8, 128), local_amax,
                                         dtype=jnp.float32)
        send_rdmas = []
        for off in range(1, N_DEV):
            peer = lax.rem(d + off, N_DEV)
            r = pltpu.make_async_remote_copy(
                src_ref=amax_ref.at[d],
                dst_ref=amax_ref.at[d],
                send_sem=amax_send_sems.at[off - 1],
                recv_sem=amax_recv_sems.at[off - 1],
                device_id=(peer,),
                device_id_type=pl.DeviceIdType.MESH,
            )
            r.start()
            send_rdmas.append(r)
        for j in range(1, N_DEV):
            src_dev = lax.rem(d + N_DEV - j, N_DEV)
            recv = pltpu.make_async_remote_copy(
                src_ref=amax_ref.at[src_dev],
                dst_ref=amax_ref.at[src_dev],
                send_sem=amax_send_sems.at[j - 1],
                recv_sem=amax_recv_sems.at[j - 1],
                device_id=(src_dev,),
                device_id_type=pl.DeviceIdType.MESH,
            )
            recv.wait_recv()
        for r in send_rdmas:
            r.wait_send()

        amax = jnp.max(amax_ref[...])
        scale = jnp.where(amax > 0.0, amax / 448.0, 1.0)
        for g in range(G):
            cols = pl.ds(g * w_g, w_g)
            q = (out_ref[:, cols] / scale).astype(jnp.float8_e4m3fn)
            out_ref[:, cols] = q.astype(jnp.float32) * scale

    return pl.pallas_call(
        body,
        out_shape=jax.ShapeDtypeStruct((m_chunk, n), jnp.float32),
        in_specs=[
            pl.BlockSpec(memory_space=pltpu.VMEM),
            pl.BlockSpec(memory_space=pltpu.VMEM),
        ],
        out_specs=pl.BlockSpec(memory_space=pltpu.VMEM),
        scratch_shapes=[
            pltpu.VMEM((2, m_chunk, w_g), jnp.float32),
            pltpu.VMEM((2, m_chunk, w_g), jnp.float32),
            pltpu.VMEM((N_DEV, 8, 128), jnp.float32),
            pltpu.SemaphoreType.DMA((2,)),
            pltpu.SemaphoreType.DMA((2,)),
            pltpu.SemaphoreType.DMA((2,)),
            pltpu.SemaphoreType.DMA((2,)),
            pltpu.SemaphoreType.DMA((N_DEV - 1,)),
            pltpu.SemaphoreType.DMA((N_DEV - 1,)),
            pltpu.SemaphoreType.REGULAR,
            pltpu.SemaphoreType.REGULAR,
        ],
        compiler_params=pltpu.CompilerParams(
            collective_id=0, vmem_limit_bytes=60 * 1024 * 1024),
    )(x, w_mat)


# device time: 713105 ns/iter; 1.9583x vs baseline; 1.0389x over previous
import jax
import jax.numpy as jnp
from jax import lax
from jax.experimental import pallas as pl
from jax.experimental.pallas import tpu as pltpu

N_DEV = 8
G = 4
N_EPOCH = G // 2


def kernel(x, w_mat):
    m, k_loc = x.shape
    _, n = w_mat.shape
    m_chunk = m // N_DEV
    w_g = n // G

    def body(x_ref, w_ref, out_ref, comm_a, comm_b, tmp_a, tmp_b, amax_ref,
             send_sems_a, recv_sems_a, send_sems_b, recv_sems_b,
             amax_send_sems, amax_recv_sems, credit_a, credit_b):
        d = lax.axis_index("i")
        left = lax.rem(d + N_DEV - 1, N_DEV)
        right = lax.rem(d + 1, N_DEV)

        barrier_sem = pltpu.get_barrier_semaphore()
        for nbr in (left, right):
            pl.semaphore_signal(barrier_sem, inc=1, device_id=(nbr,),
                                device_id_type=pl.DeviceIdType.MESH)
        pl.semaphore_wait(barrier_sem, 2)

        def partial(c, g):
            return jnp.dot(x_ref[pl.ds(c * m_chunk, m_chunk), :],
                           w_ref[:, g * w_g:(g + 1) * w_g],
                           preferred_element_type=jnp.float32)

        n_steps = N_EPOCH * (N_DEV - 1)
        t = 0
        local_amax = jnp.float32(0.0)
        for e in range(N_EPOCH):
            ga, gb = 2 * e, 2 * e + 1
            comm_a[t % 2] = partial(lax.rem(d + N_DEV - 1, N_DEV), ga)
            comm_b[t % 2] = partial(lax.rem(d + 1, N_DEV), gb)
            for s in range(N_DEV - 1):
                send_slot = t % 2
                recv_slot = (t + 1) % 2
                if t >= 1:
                    pl.semaphore_wait(credit_a, 1)
                    pl.semaphore_wait(credit_b, 1)
                rdma_a = pltpu.make_async_remote_copy(
                    src_ref=comm_a.at[send_slot],
                    dst_ref=comm_a.at[recv_slot],
                    send_sem=send_sems_a.at[send_slot],
                    recv_sem=recv_sems_a.at[recv_slot],
                    device_id=(right,),
                    device_id_type=pl.DeviceIdType.MESH,
                )
                rdma_b = pltpu.make_async_remote_copy(
                    src_ref=comm_b.at[send_slot],
                    dst_ref=comm_b.at[recv_slot],
                    send_sem=send_sems_b.at[send_slot],
                    recv_sem=recv_sems_b.at[recv_slot],
                    device_id=(left,),
                    device_id_type=pl.DeviceIdType.MESH,
                )
                rdma_a.start()
                rdma_b.start()
                ca = lax.rem(d + 2 * N_DEV - s - 2, N_DEV)
                cb = lax.rem(d + s + 2, N_DEV)
                tmp_a[...] = partial(ca, ga)
                tmp_b[...] = partial(cb, gb)
                rdma_a.wait_send()
                rdma_b.wait_send()
                if t < n_steps - 1:
                    pl.semaphore_signal(credit_a, inc=1, device_id=(left,),
                                        device_id_type=pl.DeviceIdType.MESH)
                    pl.semaphore_signal(credit_b, inc=1, device_id=(right,),
                                        device_id_type=pl.DeviceIdType.MESH)
                rdma_a.wait_recv()
                rdma_b.wait_recv()
                if s < N_DEV - 2:
                    comm_a[recv_slot] += tmp_a[...]
                    comm_b[recv_slot] += tmp_b[...]
                else:
                    res_a = jnp.maximum(comm_a[recv_slot] + tmp_a[...], 0.0)
                    res_b = jnp.maximum(comm_b[recv_slot] + tmp_b[...], 0.0)
                    local_amax = jnp.maximum(
                        local_amax,
                        jnp.maximum(jnp.max(res_a), jnp.max(res_b)))
                    out_ref[:, ga * w_g:(ga + 1) * w_g] = res_a
                    out_ref[:, gb * w_g:(gb + 1) * w_g] = res_b
                t += 1

        amax_ref[pl.ds(d, 1)] = jnp.full((1, 8, 128), local_amax,
                                         dtype=jnp.float32)
        send_rdmas = []
        for off in range(1, N_DEV):
            peer = lax.rem(d + off, N_DEV)
            r = pltpu.make_async_remote_copy(
                src_ref=amax_ref.at[d],
                dst_ref=amax_ref.at[d],
                send_sem=amax_send_sems.at[off - 1],
                recv_sem=amax_recv_sems.at[off - 1],
                device_id=(peer,),
                device_id_type=pl.DeviceIdType.MESH,
            )
            r.start()
            send_rdmas.append(r)
        for j in range(1, N_DEV):
            src_dev = lax.rem(d + N_DEV - j, N_DEV)
            recv = pltpu.make_async_remote_copy(
                src_ref=amax_ref.at[src_dev],
                dst_ref=amax_ref.at[src_dev],
                send_sem=amax_send_sems.at[j - 1],
                recv_sem=amax_recv_sems.at[j - 1],
                device_id=(src_dev,),
                device_id_type=pl.DeviceIdType.MESH,
            )
            recv.wait_recv()
        for r in send_rdmas:
            r.wait_send()

        amax = jnp.max(amax_ref[...])
        scale = jnp.where(amax > 0.0, amax / 448.0, 1.0)
        for g in range(G):
            cols = pl.ds(g * w_g, w_g)
            q = (out_ref[:, cols] / scale).astype(jnp.float8_e4m3fn)
            out_ref[:, cols] = q.astype(jnp.float32) * scale

    return pl.pallas_call(
        body,
        out_shape=jax.ShapeDtypeStruct((m_chunk, n), jnp.float32),
        in_specs=[
            pl.BlockSpec(memory_space=pltpu.VMEM),
            pl.BlockSpec(memory_space=pltpu.VMEM),
        ],
        out_specs=pl.BlockSpec(memory_space=pltpu.VMEM),
        scratch_shapes=[
            pltpu.VMEM((2, m_chunk, w_g), jnp.float32),
            pltpu.VMEM((2, m_chunk, w_g), jnp.float32),
            pltpu.VMEM((m_chunk, w_g), jnp.float32),
            pltpu.VMEM((m_chunk, w_g), jnp.float32),
            pltpu.VMEM((N_DEV, 8, 128), jnp.float32),
            pltpu.SemaphoreType.DMA((2,)),
            pltpu.SemaphoreType.DMA((2,)),
            pltpu.SemaphoreType.DMA((2,)),
            pltpu.SemaphoreType.DMA((2,)),
            pltpu.SemaphoreType.DMA((N_DEV - 1,)),
            pltpu.SemaphoreType.DMA((N_DEV - 1,)),
            pltpu.SemaphoreType.REGULAR,
            pltpu.SemaphoreType.REGULAR,
        ],
        compiler_params=pltpu.CompilerParams(
            collective_id=0, vmem_limit_bytes=100 * 1024 * 1024),
    )(x, w_mat)


# device time: 396757 ns/iter; 3.5197x vs baseline; 1.7973x over previous
import jax
import jax.numpy as jnp
from jax import lax
from jax.experimental import pallas as pl
from jax.experimental.pallas import tpu as pltpu

N_DEV = 8
G = 4
N_EPOCH = G // 2


def kernel(x, w_mat):
    m, k_loc = x.shape
    _, n = w_mat.shape
    m_chunk = m // N_DEV
    w_g = n // G

    def body(x_ref, w_ref, out_ref, comm_a, comm_b, tmp_a, tmp_b, amax_ref,
             send_sems_a, recv_sems_a, send_sems_b, recv_sems_b,
             amax_send_sems, amax_recv_sems, credit_a, credit_b):
        d = lax.axis_index("i")
        left = lax.rem(d + N_DEV - 1, N_DEV)
        right = lax.rem(d + 1, N_DEV)

        barrier_sem = pltpu.get_barrier_semaphore()
        for nbr in (left, right):
            pl.semaphore_signal(barrier_sem, inc=1, device_id=(nbr,),
                                device_id_type=pl.DeviceIdType.MESH)
        pl.semaphore_wait(barrier_sem, 2)

        def partial(c, g):
            return jnp.dot(x_ref[pl.ds(c * m_chunk, m_chunk), :],
                           w_ref[:, g * w_g:(g + 1) * w_g],
                           preferred_element_type=jnp.float32)

        n_steps = N_EPOCH * (N_DEV - 1)
        t = 0
        local_amax = jnp.float32(0.0)
        for e in range(N_EPOCH):
            ga, gb = 2 * e, 2 * e + 1
            comm_a[t % 2] = partial(
                lax.rem(d + N_DEV - 1, N_DEV), ga).astype(jnp.bfloat16)
            comm_b[t % 2] = partial(
                lax.rem(d + 1, N_DEV), gb).astype(jnp.bfloat16)
            for s in range(N_DEV - 1):
                send_slot = t % 2
                recv_slot = (t + 1) % 2
                if t >= 1:
                    pl.semaphore_wait(credit_a, 1)
                    pl.semaphore_wait(credit_b, 1)
                rdma_a = pltpu.make_async_remote_copy(
                    src_ref=comm_a.at[send_slot],
                    dst_ref=comm_a.at[recv_slot],
                    send_sem=send_sems_a.at[send_slot],
                    recv_sem=recv_sems_a.at[recv_slot],
                    device_id=(right,),
                    device_id_type=pl.DeviceIdType.MESH,
                )
                rdma_b = pltpu.make_async_remote_copy(
                    src_ref=comm_b.at[send_slot],
                    dst_ref=comm_b.at[recv_slot],
                    send_sem=send_sems_b.at[send_slot],
                    recv_sem=recv_sems_b.at[recv_slot],
                    device_id=(left,),
                    device_id_type=pl.DeviceIdType.MESH,
                )
                rdma_a.start()
                rdma_b.start()
                ca = lax.rem(d + 2 * N_DEV - s - 2, N_DEV)
                cb = lax.rem(d + s + 2, N_DEV)
                tmp_a[...] = partial(ca, ga)
                tmp_b[...] = partial(cb, gb)
                rdma_a.wait_send()
                rdma_b.wait_send()
                if t < n_steps - 1:
                    pl.semaphore_signal(credit_a, inc=1, device_id=(left,),
                                        device_id_type=pl.DeviceIdType.MESH)
                    pl.semaphore_signal(credit_b, inc=1, device_id=(right,),
                                        device_id_type=pl.DeviceIdType.MESH)
                rdma_a.wait_recv()
                rdma_b.wait_recv()
                if s < N_DEV - 2:
                    comm_a[recv_slot] = (
                        comm_a[recv_slot].astype(jnp.float32) + tmp_a[...]
                    ).astype(jnp.bfloat16)
                    comm_b[recv_slot] = (
                        comm_b[recv_slot].astype(jnp.float32) + tmp_b[...]
                    ).astype(jnp.bfloat16)
                else:
                    res_a = jnp.maximum(
                        comm_a[recv_slot].astype(jnp.float32) + tmp_a[...],
                        0.0)
                    res_b = jnp.maximum(
                        comm_b[recv_slot].astype(jnp.float32) + tmp_b[...],
                        0.0)
                    local_amax = jnp.maximum(
                        local_amax,
                        jnp.maximum(jnp.max(res_a), jnp.max(res_b)))
                    out_ref[:, ga * w_g:(ga + 1) * w_g] = res_a
                    out_ref[:, gb * w_g:(gb + 1) * w_g] = res_b
                t += 1

        amax_ref[pl.ds(d, 1)] = jnp.full((1, 8, 128), local_amax,
                                         dtype=jnp.float32)
        send_rdmas = []
        for off in range(1, N_DEV):
            peer = lax.rem(d + off, N_DEV)
            r = pltpu.make_async_remote_copy(
                src_ref=amax_ref.at[d],
                dst_ref=amax_ref.at[d],
                send_sem=amax_send_sems.at[off - 1],
                recv_sem=amax_recv_sems.at[off - 1],
                device_id=(peer,),
                device_id_type=pl.DeviceIdType.MESH,
            )
            r.start()
            send_rdmas.append(r)
        for j in range(1, N_DEV):
            src_dev = lax.rem(d + N_DEV - j, N_DEV)
            recv = pltpu.make_async_remote_copy(
                src_ref=amax_ref.at[src_dev],
                dst_ref=amax_ref.at[src_dev],
                send_sem=amax_send_sems.at[j - 1],
                recv_sem=amax_recv_sems.at[j - 1],
                device_id=(src_dev,),
                device_id_type=pl.DeviceIdType.MESH,
            )
            recv.wait_recv()
        for r in send_rdmas:
            r.wait_send()

        amax = jnp.max(amax_ref[...])
        scale = jnp.where(amax > 0.0, amax / 448.0, 1.0)
        for g in range(G):
            cols = pl.ds(g * w_g, w_g)
            q = (out_ref[:, cols] / scale).astype(jnp.float8_e4m3fn)
            out_ref[:, cols] = q.astype(jnp.float32) * scale

    return pl.pallas_call(
        body,
        out_shape=jax.ShapeDtypeStruct((m_chunk, n), jnp.float32),
        in_specs=[
            pl.BlockSpec(memory_space=pltpu.VMEM),
            pl.BlockSpec(memory_space=pltpu.VMEM),
        ],
        out_specs=pl.BlockSpec(memory_space=pltpu.VMEM),
        scratch_shapes=[
            pltpu.VMEM((2, m_chunk, w_g), jnp.bfloat16),
            pltpu.VMEM((2, m_chunk, w_g), jnp.bfloat16),
            pltpu.VMEM((m_chunk, w_g), jnp.float32),
            pltpu.VMEM((m_chunk, w_g), jnp.float32),
            pltpu.VMEM((N_DEV, 8, 128), jnp.float32),
            pltpu.SemaphoreType.DMA((2,)),
            pltpu.SemaphoreType.DMA((2,)),
            pltpu.SemaphoreType.DMA((2,)),
            pltpu.SemaphoreType.DMA((2,)),
            pltpu.SemaphoreType.DMA((N_DEV - 1,)),
            pltpu.SemaphoreType.DMA((N_DEV - 1,)),
            pltpu.SemaphoreType.REGULAR,
            pltpu.SemaphoreType.REGULAR,
        ],
        compiler_params=pltpu.CompilerParams(
            collective_id=0, vmem_limit_bytes=100 * 1024 * 1024),
    )(x, w_mat)


# device time: 356114 ns/iter; 3.9214x vs baseline; 1.1141x over previous
import jax
import jax.numpy as jnp
from jax import lax
from jax.experimental import pallas as pl
from jax.experimental.pallas import tpu as pltpu

N_DEV = 8
N_EPOCH = 2
W_SUB = 1024
N_STEPS = N_EPOCH * (N_DEV - 1)


def kernel(x, w_mat):
    m, k_loc = x.shape
    _, n = w_mat.shape
    m_chunk = m // N_DEV
    w_e = n // N_EPOCH

    def body(x_ref, w_ref, out_ref,
             comm_a0, comm_b0, comm_a1, comm_b1,
             tmp_a0, tmp_b0, tmp_a1, tmp_b1, amax_ref,
             ss_a0, rs_a0, ss_b0, rs_b0, ss_a1, rs_a1, ss_b1, rs_b1,
             amax_send_sems, amax_recv_sems,
             cr_a0, cr_b0, cr_a1, cr_b1):
        d = lax.axis_index("i")
        left = lax.rem(d + N_DEV - 1, N_DEV)
        right = lax.rem(d + 1, N_DEV)

        barrier_sem = pltpu.get_barrier_semaphore()
        for nbr in (left, right):
            pl.semaphore_signal(barrier_sem, inc=1, device_id=(nbr,),
                                device_id_type=pl.DeviceIdType.MESH)
        pl.semaphore_wait(barrier_sem, 2)

        subs = [
            (comm_a0, tmp_a0, ss_a0, rs_a0, cr_a0, True, 0),
            (comm_b0, tmp_b0, ss_b0, rs_b0, cr_b0, False, 0),
            (comm_a1, tmp_a1, ss_a1, rs_a1, cr_a1, True, 1),
            (comm_b1, tmp_b1, ss_b1, rs_b1, cr_b1, False, 1),
        ]

        def cols(e, is_cw, j):
            base = e * w_e + (0 if is_cw else w_e // 2) + j * W_SUB
            return slice(base, base + W_SUB)

        def x_slice(c):
            return x_ref[pl.ds(c * m_chunk, m_chunk), :]

        def chunk_recv(is_cw, s):
            if is_cw:
                return lax.rem(d + 2 * N_DEV - s - 2, N_DEV)
            return lax.rem(d + s + 2, N_DEV)

        c_init_a = lax.rem(d + N_DEV - 1, N_DEV)
        c_init_b = lax.rem(d + 1, N_DEV)

        def start(sub, t):
            comm, _, ss, rs, cr, is_cw, _ = sub
            if t >= 1:
                pl.semaphore_wait(cr, 1)
            r = pltpu.make_async_remote_copy(
                src_ref=comm.at[t % 2],
                dst_ref=comm.at[(t + 1) % 2],
                send_sem=ss.at[t % 2],
                recv_sem=rs.at[(t + 1) % 2],
                device_id=(right if is_cw else left,),
                device_id_type=pl.DeviceIdType.MESH,
            )
            r.start()
            return r

        local_amax = jnp.float32(0.0)

        xia = x_slice(c_init_a)
        xib = x_slice(c_init_b)
        rdmas = {}
        for si, sub in enumerate(subs):
            comm, _, _, _, _, is_cw, j = sub
            xs = xia if is_cw else xib
            comm[0] = jnp.dot(
                xs, w_ref[:, cols(0, is_cw, j)],
                preferred_element_type=jnp.float32).astype(jnp.bfloat16)
            rdmas[si] = start(sub, 0)

        for t in range(1, N_STEPS + 1):
            s_prev = (t - 1) % (N_DEV - 1)
            e_prev = (t - 1) // (N_DEV - 1)
            recv_slot = t % 2
            xa = x_slice(chunk_recv(True, s_prev))
            xb = x_slice(chunk_recv(False, s_prev))
            if t % (N_DEV - 1) == 0 and t < N_STEPS:
                xia = x_slice(c_init_a)
                xib = x_slice(c_init_b)
            for si, sub in enumerate(subs):
                comm, tmp, _, _, cr, is_cw, j = sub
                xs = xa if is_cw else xb
                tmp[...] = jnp.dot(
                    xs, w_ref[:, cols(e_prev, is_cw, j)],
                    preferred_element_type=jnp.float32)
                r = rdmas[si]
                r.wait_recv()
                if s_prev < N_DEV - 2:
                    comm[recv_slot] = (
                        comm[recv_slot].astype(jnp.float32) + tmp[...]
                    ).astype(jnp.bfloat16)
                else:
                    res = jnp.maximum(
                        comm[recv_slot].astype(jnp.float32) + tmp[...], 0.0)
                    local_amax = jnp.maximum(local_amax, jnp.max(res))
                    out_ref[:, cols(e_prev, is_cw, j)] = res
                r.wait_send()
                if t <= N_STEPS - 1:
                    pl.semaphore_signal(
                        cr, inc=1,
                        device_id=(left if is_cw else right,),
                        device_id_type=pl.DeviceIdType.MESH)
                if t < N_STEPS:
                    if t % (N_DEV - 1) == 0:
                        xs_i = xia if is_cw else xib
                        comm[t % 2] = jnp.dot(
                            xs_i, w_ref[:, cols(t // (N_DEV - 1), is_cw, j)],
                            preferred_element_type=jnp.float32,
                        ).astype(jnp.bfloat16)
                    rdmas[si] = start(sub, t)

        amax_ref[pl.ds(d, 1)] = jnp.full((1, 8, 128), local_amax,
                                         dtype=jnp.float32)
        send_rdmas = []
        for off in range(1, N_DEV):
            peer = lax.rem(d + off, N_DEV)
            r = pltpu.make_async_remote_copy(
                src_ref=amax_ref.at[d],
                dst_ref=amax_ref.at[d],
                send_sem=amax_send_sems.at[off - 1],
                recv_sem=amax_recv_sems.at[off - 1],
                device_id=(peer,),
                device_id_type=pl.DeviceIdType.MESH,
            )
            r.start()
            send_rdmas.append(r)
        for jj in range(1, N_DEV):
            src_dev = lax.rem(d + N_DEV - jj, N_DEV)
            recv = pltpu.make_async_remote_copy(
                src_ref=amax_ref.at[src_dev],
                dst_ref=amax_ref.at[src_dev],
                send_sem=amax_send_sems.at[jj - 1],
                recv_sem=amax_recv_sems.at[jj - 1],
                device_id=(src_dev,),
                device_id_type=pl.DeviceIdType.MESH,
            )
            recv.wait_recv()
        for r in send_rdmas:
            r.wait_send()

        amax = jnp.max(amax_ref[...])
        scale = jnp.where(amax > 0.0, amax / 448.0, 1.0)
        for g in range(n // W_SUB):
            csl = pl.ds(g * W_SUB, W_SUB)
            q = (out_ref[:, csl] / scale).astype(jnp.float8_e4m3fn)
            out_ref[:, csl] = q.astype(jnp.float32) * scale

    comm_shape = pltpu.VMEM((2, m_chunk, W_SUB), jnp.bfloat16)
    tmp_shape = pltpu.VMEM((m_chunk, W_SUB), jnp.float32)
    return pl.pallas_call(
        body,
        out_shape=jax.ShapeDtypeStruct((m_chunk, n), jnp.float32),
        in_specs=[
            pl.BlockSpec(memory_space=pltpu.VMEM),
            pl.BlockSpec(memory_space=pltpu.VMEM),
        ],
        out_specs=pl.BlockSpec(memory_space=pltpu.VMEM),
        scratch_shapes=[
            comm_shape, comm_shape, comm_shape, comm_shape,
            tmp_shape, tmp_shape, tmp_shape, tmp_shape,
            pltpu.VMEM((N_DEV, 8, 128), jnp.float32),
            pltpu.SemaphoreType.DMA((2,)),
            pltpu.SemaphoreType.DMA((2,)),
            pltpu.SemaphoreType.DMA((2,)),
            pltpu.SemaphoreType.DMA((2,)),
            pltpu.SemaphoreType.DMA((2,)),
            pltpu.SemaphoreType.DMA((2,)),
            pltpu.SemaphoreType.DMA((2,)),
            pltpu.SemaphoreType.DMA((2,)),
            pltpu.SemaphoreType.DMA((N_DEV - 1,)),
            pltpu.SemaphoreType.DMA((N_DEV - 1,)),
            pltpu.SemaphoreType.REGULAR,
            pltpu.SemaphoreType.REGULAR,
            pltpu.SemaphoreType.REGULAR,
            pltpu.SemaphoreType.REGULAR,
        ],
        compiler_params=pltpu.CompilerParams(
            collective_id=0, vmem_limit_bytes=100 * 1024 * 1024),
    )(x, w_mat)
